# baseline (speedup 1.0000x reference)
import functools

import jax
import jax.numpy as jnp
from jax import lax
from jax.experimental import pallas as pl
from jax.experimental.pallas import tpu as pltpu
from jax.experimental.pallas import tpu_sc as plsc

_R = 0.1
_MARGIN = 0.15
_N = 4096
_NW = 16
_PPT = _N // _NW
_NBINS = 64
_BI = 1024
_CH = 512
_CIN = 64
_COUT = 64

_mesh = plsc.VectorSubcoreMesh(core_axis_name="c", subcore_axis_name="s",
                               num_cores=1)


@functools.partial(
    pl.kernel,
    out_type=(
        jax.ShapeDtypeStruct((_N, 16), jnp.float32),
        jax.ShapeDtypeStruct((_N, _CIN), jnp.float32),
        jax.ShapeDtypeStruct((_N,), jnp.int32),
    ),
    mesh=_mesh,
    scratch_types=[
        pltpu.VMEM((_PPT,), jnp.float32),
        pltpu.VMEM((_PPT,), jnp.int32),
        pltpu.VMEM((16 * _NBINS,), jnp.int32),
        pltpu.VMEM((_NBINS,), jnp.int32),
        pltpu.VMEM((_NW, _NBINS), jnp.int32),
        pltpu.VMEM((_NBINS,), jnp.int32),
        pltpu.VMEM((_PPT,), jnp.int32),
        pltpu.VMEM((_PPT, 16), jnp.float32),
        pltpu.VMEM((_PPT, _CIN), jnp.float32),
        pltpu.VMEM_SHARED((_NW, _NBINS), jnp.int32),
        pltpu.SemaphoreType.DMA,
    ],
    compiler_params=pltpu.CompilerParams(needs_layout_passes=False, use_tc_tiling_on_sc=False),
)
def _binperm(x_hbm, pr_hbm, dT_hbm,
             prs_hbm, dTs_hbm, rank_hbm,
             x_v, bins_v, lhist_v, hist_v, hists_v, off_v, dst_v, prr_v, dTr_v,
             hists_sh, sem):
    wid = lax.axis_index("s")
    base = wid * _PPT

    pltpu.sync_copy(x_hbm.at[pl.ds(base, _PPT)], x_v)

    zero16 = jnp.zeros((16,), jnp.int32)
    for k in range(16 * _NBINS // 16):
        lhist_v[pl.ds(k * 16, 16)] = zero16
    ones16 = jnp.ones((16,), jnp.int32)
    lane16 = lax.iota(jnp.int32, 16) * _NBINS
    for k in range(_PPT // 16):
        xv = x_v[pl.ds(k * 16, 16)]
        bv = (xv * float(_NBINS)).astype(jnp.int32)
        bv = jnp.minimum(jnp.maximum(bv, 0), _NBINS - 1)
        bins_v[pl.ds(k * 16, 16)] = bv
        plsc.addupdate_scatter(lhist_v, [lane16 + bv], ones16)
    for c in range(_NBINS // 16):
        acc = zero16
        for l in range(16):
            acc = acc + lhist_v[pl.ds(l * _NBINS + c * 16, 16)]
        hist_v[pl.ds(c * 16, 16)] = acc

    pltpu.sync_copy(hist_v, hists_sh.at[wid])
    plsc.subcore_barrier()
    pltpu.sync_copy(hists_sh, hists_v)

    carry = jnp.zeros((), jnp.int32)
    for k in range(_NBINS // 16):
        tot = jnp.zeros((16,), jnp.int32)
        pre = jnp.zeros((16,), jnp.int32)
        for t in range(_NW):
            h = hists_v[t, pl.ds(k * 16, 16)]
            tot = tot + h
            sel = jnp.asarray(t < wid, jnp.int32)
            pre = pre + h * sel
        incl = plsc.cumsum(tot)
        excl = incl - tot
        off_v[pl.ds(k * 16, 16)] = excl + pre + carry
        carry = carry + jnp.sum(tot)

    lane0 = lax.iota(jnp.int32, 16) == 0

    def place(p, _):
        pv = jnp.full((16,), p, jnp.int32)
        bv = plsc.load_gather(bins_v, [pv])
        ov = plsc.load_gather(off_v, [bv])
        plsc.store_scatter(off_v, [bv], ov + 1, mask=lane0)
        plsc.store_scatter(dst_v, [pv], ov, mask=lane0)
        return 0
    lax.fori_loop(0, _PPT, place, 0)

    pltpu.sync_copy(pr_hbm.at[pl.ds(base, _PPT)], prr_v)
    pltpu.sync_copy(dT_hbm.at[pl.ds(base, _PPT)], dTr_v)
    pltpu.async_copy(prr_v, prs_hbm.at[dst_v], sem).wait()
    pltpu.async_copy(dTr_v, dTs_hbm.at[dst_v], sem).wait()
    pltpu.sync_copy(dst_v, rank_hbm.at[pl.ds(base, _PPT)])


def _windowed_body(prb_ref, posT_ref, xc_ref, xr_ref, p2c_ref, p2r_ref,
                   dTs_ref, w2T_ref, bias_ref, out_ref, wdT_ref):
    @pl.when(pl.program_id(0) == 0)
    def _():
        wdT_ref[...] = jnp.dot(dTs_ref[...], w2T_ref[...],
                               preferred_element_type=jnp.float32,
                               precision=jax.lax.Precision.HIGHEST)

    pos_b = prb_ref[...]
    xblk = xc_ref[...]
    p2c = p2c_ref[...]
    bmin = jnp.floor((jnp.min(xblk) - _MARGIN) * float(_NBINS))
    bmax = jnp.floor((jnp.max(xblk) + _MARGIN) * float(_NBINS))
    ball = jnp.floor(xr_ref[...] * float(_NBINS))
    jlo = jnp.sum((ball < bmin).astype(jnp.int32))
    jhi = jnp.sum((ball <= bmax).astype(jnp.int32))

    out_ref[...] = jnp.zeros_like(out_ref) + bias_ref[...]
    for k in range(_N // _CH):
        js = k * _CH

        @pl.when((js + _CH > jlo) & (js < jhi))
        def _():
            cross = jax.lax.dot_general(
                pos_b, posT_ref[:, pl.ds(js, _CH)], (((1,), (0,)), ((), ())),
                preferred_element_type=jnp.float32)
            d2 = jnp.maximum(
                (p2c + p2r_ref[:, pl.ds(js, _CH)]) - 2.0 * cross, 0.0)
            dist = jnp.sqrt(d2)
            t = jnp.maximum(1.0 - dist * (1.0 / _R), 0.0)
            w = t * t * t
            out_ref[...] += jax.lax.dot_general(
                w, wdT_ref[pl.ds(js, _CH), :], (((1,), (0,)), ((), ())),
                preferred_element_type=jnp.float32)


def _windowed(pos4s, xcol, p2s, dTs, w2T, bias_row):
    return pl.pallas_call(
        _windowed_body,
        grid=(_N // _BI,),
        in_specs=[
            pl.BlockSpec((_BI, 4), lambda i: (i, 0)),
            pl.BlockSpec((4, _N), lambda i: (0, 0)),
            pl.BlockSpec((_BI, 1), lambda i: (i, 0)),
            pl.BlockSpec((1, _N), lambda i: (0, 0)),
            pl.BlockSpec((_BI, 1), lambda i: (i, 0)),
            pl.BlockSpec((1, _N), lambda i: (0, 0)),
            pl.BlockSpec((_N, _CIN), lambda i: (0, 0)),
            pl.BlockSpec((_CIN, _COUT), lambda i: (0, 0)),
            pl.BlockSpec((1, _COUT), lambda i: (0, 0)),
        ],
        out_specs=pl.BlockSpec((_BI, _COUT), lambda i: (i, 0)),
        out_shape=jax.ShapeDtypeStruct((_N, _COUT), jnp.float32),
        scratch_shapes=[pltpu.VMEM((_N, _COUT), jnp.float32)],
    )(pos4s, pos4s.T, xcol, xcol.reshape(1, _N), p2s, p2s.reshape(1, _N),
      dTs, w2T, bias_row)


@functools.partial(
    pl.kernel,
    out_type=jax.ShapeDtypeStruct((_N, _COUT), jnp.float32),
    mesh=_mesh,
    scratch_types=[
        pltpu.VMEM((_PPT,), jnp.int32),
        pltpu.VMEM((_PPT, _COUT), jnp.float32),
        pltpu.SemaphoreType.DMA,
    ],
    compiler_params=pltpu.CompilerParams(needs_layout_passes=False, use_tc_tiling_on_sc=False),
)
def _gatherback(outs_hbm, rank_hbm, outT_hbm, idx_v, rows_v, sem):
    wid = lax.axis_index("s")
    base = wid * _PPT
    pltpu.sync_copy(rank_hbm.at[pl.ds(base, _PPT)], idx_v)
    pltpu.async_copy(outs_hbm.at[idx_v], rows_v, sem).wait()
    pltpu.sync_copy(rows_v, outT_hbm.at[pl.ds(base, _PPT)])


def kernel(locs, data, density, weight, bias):
    B, N, _ = locs.shape
    pos = locs[0, :, :3]
    x = pos[:, 0]
    zcol = jnp.zeros((N, 1), jnp.float32)
    p2 = jnp.sum(pos * pos, axis=1, keepdims=True)
    pr = jnp.concatenate(
        [pos, zcol, p2, jnp.zeros((N, 11), jnp.float32)], axis=1)
    dT = (data[0] / density.reshape(1, N)).T
    w2T = weight[:, :, 0].T
    bias_row = bias.reshape(1, _COUT)

    prs, dTs, rank = _binperm(x, pr, dT)
    pos4s = prs[:, :4]
    outs = _windowed(pos4s, prs[:, 0:1], prs[:, 4:5], dTs, w2T, bias_row)
    outT = _gatherback(outs, rank)
    return outT.T.reshape(B, _COUT, N)


# speedup vs baseline: 1.0023x; 1.0023x over previous
import functools

import jax
import jax.numpy as jnp
from jax import lax
from jax.experimental import pallas as pl
from jax.experimental.pallas import tpu as pltpu
from jax.experimental.pallas import tpu_sc as plsc

_R = 0.1
_MARGIN = 0.15
_N = 4096
_NW = 16
_PPT = _N // _NW
_NBINS = 64
_BI = 512
_CH = 512
_CIN = 64
_COUT = 64

_mesh = plsc.VectorSubcoreMesh(core_axis_name="c", subcore_axis_name="s",
                               num_cores=1)


@functools.partial(
    pl.kernel,
    out_type=(
        jax.ShapeDtypeStruct((_N, 16), jnp.float32),
        jax.ShapeDtypeStruct((_N, _CIN), jnp.float32),
        jax.ShapeDtypeStruct((_N,), jnp.int32),
    ),
    mesh=_mesh,
    scratch_types=[
        pltpu.VMEM((_PPT,), jnp.float32),
        pltpu.VMEM((_PPT,), jnp.int32),
        pltpu.VMEM((16 * _NBINS,), jnp.int32),
        pltpu.VMEM((_NBINS,), jnp.int32),
        pltpu.VMEM((_NW, _NBINS), jnp.int32),
        pltpu.VMEM((_NBINS,), jnp.int32),
        pltpu.VMEM((_PPT,), jnp.int32),
        pltpu.VMEM((_PPT, 16), jnp.float32),
        pltpu.VMEM((_PPT, _CIN), jnp.float32),
        pltpu.VMEM_SHARED((_NW, _NBINS), jnp.int32),
        pltpu.SemaphoreType.DMA,
    ],
    compiler_params=pltpu.CompilerParams(needs_layout_passes=False, use_tc_tiling_on_sc=False),
)
def _binperm(x_hbm, pr_hbm, dT_hbm,
             prs_hbm, dTs_hbm, rank_hbm,
             x_v, bins_v, lhist_v, hist_v, hists_v, off_v, dst_v, prr_v, dTr_v,
             hists_sh, sem):
    wid = lax.axis_index("s")
    base = wid * _PPT

    pltpu.sync_copy(x_hbm.at[pl.ds(base, _PPT)], x_v)

    zero16 = jnp.zeros((16,), jnp.int32)
    for k in range(16 * _NBINS // 16):
        lhist_v[pl.ds(k * 16, 16)] = zero16
    ones16 = jnp.ones((16,), jnp.int32)
    lane16 = lax.iota(jnp.int32, 16) * _NBINS
    for k in range(_PPT // 16):
        xv = x_v[pl.ds(k * 16, 16)]
        bv = (xv * float(_NBINS)).astype(jnp.int32)
        bv = jnp.minimum(jnp.maximum(bv, 0), _NBINS - 1)
        bins_v[pl.ds(k * 16, 16)] = bv
        plsc.addupdate_scatter(lhist_v, [lane16 + bv], ones16)
    for c in range(_NBINS // 16):
        acc = zero16
        for l in range(16):
            acc = acc + lhist_v[pl.ds(l * _NBINS + c * 16, 16)]
        hist_v[pl.ds(c * 16, 16)] = acc

    pltpu.sync_copy(hist_v, hists_sh.at[wid])
    plsc.subcore_barrier()
    pltpu.sync_copy(hists_sh, hists_v)

    carry = jnp.zeros((), jnp.int32)
    for k in range(_NBINS // 16):
        tot = jnp.zeros((16,), jnp.int32)
        pre = jnp.zeros((16,), jnp.int32)
        for t in range(_NW):
            h = hists_v[t, pl.ds(k * 16, 16)]
            tot = tot + h
            sel = jnp.asarray(t < wid, jnp.int32)
            pre = pre + h * sel
        incl = plsc.cumsum(tot)
        excl = incl - tot
        off_v[pl.ds(k * 16, 16)] = excl + pre + carry
        carry = carry + jnp.sum(tot)

    lane0 = lax.iota(jnp.int32, 16) == 0

    def place(p, _):
        pv = jnp.full((16,), p, jnp.int32)
        bv = plsc.load_gather(bins_v, [pv])
        ov = plsc.load_gather(off_v, [bv])
        plsc.store_scatter(off_v, [bv], ov + 1, mask=lane0)
        plsc.store_scatter(dst_v, [pv], ov, mask=lane0)
        return 0
    lax.fori_loop(0, _PPT, place, 0)

    pltpu.sync_copy(pr_hbm.at[pl.ds(base, _PPT)], prr_v)
    pltpu.sync_copy(dT_hbm.at[pl.ds(base, _PPT)], dTr_v)
    pltpu.async_copy(prr_v, prs_hbm.at[dst_v], sem).wait()
    pltpu.async_copy(dTr_v, dTs_hbm.at[dst_v], sem).wait()
    pltpu.sync_copy(dst_v, rank_hbm.at[pl.ds(base, _PPT)])


def _windowed_body(prb_ref, posT_ref, xc_ref, xr_ref, p2c_ref, p2r_ref,
                   dTs_ref, w2T_ref, bias_ref, out_ref, wdT_ref):
    @pl.when(pl.program_id(0) == 0)
    def _():
        wdT_ref[...] = jnp.dot(dTs_ref[...], w2T_ref[...],
                               preferred_element_type=jnp.float32,
                               precision=jax.lax.Precision.HIGHEST)

    pos_b = prb_ref[...]
    xblk = xc_ref[...]
    p2c = p2c_ref[...]
    bmin = jnp.floor((jnp.min(xblk) - _MARGIN) * float(_NBINS))
    bmax = jnp.floor((jnp.max(xblk) + _MARGIN) * float(_NBINS))
    ball = jnp.floor(xr_ref[...] * float(_NBINS))
    jlo = jnp.sum((ball < bmin).astype(jnp.int32))
    jhi = jnp.sum((ball <= bmax).astype(jnp.int32))

    out_ref[...] = jnp.zeros_like(out_ref) + bias_ref[...]
    for k in range(_N // _CH):
        js = k * _CH

        @pl.when((js + _CH > jlo) & (js < jhi))
        def _():
            cross = jax.lax.dot_general(
                pos_b, posT_ref[:, pl.ds(js, _CH)], (((1,), (0,)), ((), ())),
                preferred_element_type=jnp.float32)
            d2 = jnp.maximum(
                (p2c + p2r_ref[:, pl.ds(js, _CH)]) - 2.0 * cross, 0.0)
            dist = jnp.sqrt(d2)
            t = jnp.maximum(1.0 - dist * (1.0 / _R), 0.0)
            w = t * t * t
            out_ref[...] += jax.lax.dot_general(
                w, wdT_ref[pl.ds(js, _CH), :], (((1,), (0,)), ((), ())),
                preferred_element_type=jnp.float32)


def _windowed(pos4s, xcol, p2s, dTs, w2T, bias_row):
    return pl.pallas_call(
        _windowed_body,
        grid=(_N // _BI,),
        in_specs=[
            pl.BlockSpec((_BI, 4), lambda i: (i, 0)),
            pl.BlockSpec((4, _N), lambda i: (0, 0)),
            pl.BlockSpec((_BI, 1), lambda i: (i, 0)),
            pl.BlockSpec((1, _N), lambda i: (0, 0)),
            pl.BlockSpec((_BI, 1), lambda i: (i, 0)),
            pl.BlockSpec((1, _N), lambda i: (0, 0)),
            pl.BlockSpec((_N, _CIN), lambda i: (0, 0)),
            pl.BlockSpec((_CIN, _COUT), lambda i: (0, 0)),
            pl.BlockSpec((1, _COUT), lambda i: (0, 0)),
        ],
        out_specs=pl.BlockSpec((_BI, _COUT), lambda i: (i, 0)),
        out_shape=jax.ShapeDtypeStruct((_N, _COUT), jnp.float32),
        scratch_shapes=[pltpu.VMEM((_N, _COUT), jnp.float32)],
    )(pos4s, pos4s.T, xcol, xcol.reshape(1, _N), p2s, p2s.reshape(1, _N),
      dTs, w2T, bias_row)


@functools.partial(
    pl.kernel,
    out_type=jax.ShapeDtypeStruct((_N, _COUT), jnp.float32),
    mesh=_mesh,
    scratch_types=[
        pltpu.VMEM((_PPT,), jnp.int32),
        pltpu.VMEM((_PPT, _COUT), jnp.float32),
        pltpu.SemaphoreType.DMA,
    ],
    compiler_params=pltpu.CompilerParams(needs_layout_passes=False, use_tc_tiling_on_sc=False),
)
def _gatherback(outs_hbm, rank_hbm, outT_hbm, idx_v, rows_v, sem):
    wid = lax.axis_index("s")
    base = wid * _PPT
    pltpu.sync_copy(rank_hbm.at[pl.ds(base, _PPT)], idx_v)
    pltpu.async_copy(outs_hbm.at[idx_v], rows_v, sem).wait()
    pltpu.sync_copy(rows_v, outT_hbm.at[pl.ds(base, _PPT)])


def kernel(locs, data, density, weight, bias):
    B, N, _ = locs.shape
    pos = locs[0, :, :3]
    x = pos[:, 0]
    zcol = jnp.zeros((N, 1), jnp.float32)
    p2 = jnp.sum(pos * pos, axis=1, keepdims=True)
    pr = jnp.concatenate(
        [pos, zcol, p2, jnp.zeros((N, 11), jnp.float32)], axis=1)
    dT = (data[0] / density.reshape(1, N)).T
    w2T = weight[:, :, 0].T
    bias_row = bias.reshape(1, _COUT)

    prs, dTs, rank = _binperm(x, pr, dT)
    pos4s = prs[:, :4]
    outs = _windowed(pos4s, prs[:, 0:1], prs[:, 4:5], dTs, w2T, bias_row)
    outT = _gatherback(outs, rank)
    return outT.T.reshape(B, _COUT, N)


# speedup vs baseline: 1.0663x; 1.0639x over previous
import functools

import jax
import jax.numpy as jnp
from jax import lax
from jax.experimental import pallas as pl
from jax.experimental.pallas import tpu as pltpu
from jax.experimental.pallas import tpu_sc as plsc

_R = 0.1
_MARGIN = 0.15
_N = 4096
_NW = 16
_PPT = _N // _NW
_NBINS = 64
_BI = 512
_CH = 512
_CIN = 64
_COUT = 64

_mesh = plsc.VectorSubcoreMesh(core_axis_name="c", subcore_axis_name="s",
                               num_cores=1)


@functools.partial(
    pl.kernel,
    out_type=(
        jax.ShapeDtypeStruct((_N, 16), jnp.float32),
        jax.ShapeDtypeStruct((_N, _CIN), jnp.float32),
        jax.ShapeDtypeStruct((_N,), jnp.int32),
    ),
    mesh=_mesh,
    scratch_types=[
        pltpu.VMEM((_PPT,), jnp.float32),
        pltpu.VMEM((_PPT,), jnp.int32),
        pltpu.VMEM((16 * _NBINS,), jnp.int32),
        pltpu.VMEM((_NBINS,), jnp.int32),
        pltpu.VMEM((_NW, _NBINS), jnp.int32),
        pltpu.VMEM((_NBINS,), jnp.int32),
        pltpu.VMEM((_PPT,), jnp.int32),
        pltpu.VMEM((_PPT, 16), jnp.float32),
        pltpu.VMEM((_PPT, _CIN), jnp.float32),
        pltpu.VMEM_SHARED((_NW, _NBINS), jnp.int32),
        pltpu.SemaphoreType.DMA,
    ],
    compiler_params=pltpu.CompilerParams(needs_layout_passes=False, use_tc_tiling_on_sc=False),
)
def _binperm(x_hbm, pr_hbm, dT_hbm,
             prs_hbm, dTs_hbm, rank_hbm,
             x_v, bins_v, lhist_v, hist_v, hists_v, off_v, dst_v, prr_v, dTr_v,
             hists_sh, sem):
    wid = lax.axis_index("s")
    base = wid * _PPT

    pltpu.sync_copy(x_hbm.at[pl.ds(base, _PPT)], x_v)

    zero16 = jnp.zeros((16,), jnp.int32)
    for k in range(16 * _NBINS // 16):
        lhist_v[pl.ds(k * 16, 16)] = zero16
    ones16 = jnp.ones((16,), jnp.int32)
    lane16 = lax.iota(jnp.int32, 16) * _NBINS
    for k in range(_PPT // 16):
        xv = x_v[pl.ds(k * 16, 16)]
        bv = (xv * float(_NBINS)).astype(jnp.int32)
        bv = jnp.minimum(jnp.maximum(bv, 0), _NBINS - 1)
        bins_v[pl.ds(k * 16, 16)] = bv
        plsc.addupdate_scatter(lhist_v, [lane16 + bv], ones16)
    for c in range(_NBINS // 16):
        acc = zero16
        for l in range(16):
            acc = acc + lhist_v[pl.ds(l * _NBINS + c * 16, 16)]
        hist_v[pl.ds(c * 16, 16)] = acc

    pltpu.sync_copy(hist_v, hists_sh.at[wid])
    plsc.subcore_barrier()
    pltpu.sync_copy(hists_sh, hists_v)

    carry = jnp.zeros((), jnp.int32)
    for k in range(_NBINS // 16):
        tot = jnp.zeros((16,), jnp.int32)
        pre = jnp.zeros((16,), jnp.int32)
        for t in range(_NW):
            h = hists_v[t, pl.ds(k * 16, 16)]
            tot = tot + h
            sel = jnp.asarray(t < wid, jnp.int32)
            pre = pre + h * sel
        incl = plsc.cumsum(tot)
        excl = incl - tot
        off_v[pl.ds(k * 16, 16)] = excl + pre + carry
        carry = carry + jnp.sum(tot)

    lane0 = lax.iota(jnp.int32, 16) == 0

    def place(p, _):
        pv = jnp.full((16,), p, jnp.int32)
        bv = plsc.load_gather(bins_v, [pv])
        ov = plsc.load_gather(off_v, [bv])
        plsc.store_scatter(off_v, [bv], ov + 1, mask=lane0)
        plsc.store_scatter(dst_v, [pv], ov, mask=lane0)
        return 0
    lax.fori_loop(0, _PPT, place, 0)

    pltpu.sync_copy(pr_hbm.at[pl.ds(base, _PPT)], prr_v)
    pltpu.sync_copy(dT_hbm.at[pl.ds(base, _PPT)], dTr_v)
    pltpu.async_copy(prr_v, prs_hbm.at[dst_v], sem).wait()
    pltpu.async_copy(dTr_v, dTs_hbm.at[dst_v], sem).wait()
    pltpu.sync_copy(dst_v, rank_hbm.at[pl.ds(base, _PPT)])


def _windowed_body(prb_ref, posT_ref, xc_ref, xr_ref, p2c_ref, p2r_ref,
                   dTs_ref, w2T_ref, bias_ref, out_ref, wdT_ref):
    @pl.when(pl.program_id(0) == 0)
    def _():
        wdT_ref[...] = jnp.dot(dTs_ref[...], w2T_ref[...],
                               preferred_element_type=jnp.float32,
                               precision=jax.lax.Precision.HIGHEST)

    pos_b = prb_ref[...]
    xblk = xc_ref[...]
    p2c = p2c_ref[...]
    bmin = jnp.floor((jnp.min(xblk) - _MARGIN) * float(_NBINS))
    bmax = jnp.floor((jnp.max(xblk) + _MARGIN) * float(_NBINS))
    ball = jnp.floor(xr_ref[...] * float(_NBINS))
    jlo = jnp.sum((ball < bmin).astype(jnp.int32))
    jhi = jnp.sum((ball <= bmax).astype(jnp.int32))

    out_ref[...] = jnp.zeros_like(out_ref) + bias_ref[...]
    for k in range(_N // _CH):
        js = k * _CH

        @pl.when((js + _CH > jlo) & (js < jhi))
        def _():
            cross = jax.lax.dot_general(
                pos_b, posT_ref[:, pl.ds(js, _CH)], (((1,), (0,)), ((), ())),
                preferred_element_type=jnp.float32)
            d2 = jnp.maximum(
                (p2c + p2r_ref[:, pl.ds(js, _CH)]) - 2.0 * cross, 0.0)
            dist = jnp.sqrt(d2)
            t = jnp.maximum(1.0 - dist * (1.0 / _R), 0.0)
            w = t * t * t
            out_ref[...] += jax.lax.dot_general(
                w, wdT_ref[pl.ds(js, _CH), :], (((1,), (0,)), ((), ())),
                preferred_element_type=jnp.float32)


def _windowed(pos4s, xcol, p2s, dTs, w2T, bias_row):
    return pl.pallas_call(
        _windowed_body,
        grid=(_N // _BI,),
        in_specs=[
            pl.BlockSpec((_BI, 4), lambda i: (i, 0)),
            pl.BlockSpec((4, _N), lambda i: (0, 0)),
            pl.BlockSpec((_BI, 1), lambda i: (i, 0)),
            pl.BlockSpec((1, _N), lambda i: (0, 0)),
            pl.BlockSpec((_BI, 1), lambda i: (i, 0)),
            pl.BlockSpec((1, _N), lambda i: (0, 0)),
            pl.BlockSpec((_N, _CIN), lambda i: (0, 0)),
            pl.BlockSpec((_CIN, _COUT), lambda i: (0, 0)),
            pl.BlockSpec((1, _COUT), lambda i: (0, 0)),
        ],
        out_specs=pl.BlockSpec((_BI, _COUT), lambda i: (i, 0)),
        out_shape=jax.ShapeDtypeStruct((_N, _COUT), jnp.float32),
        scratch_shapes=[pltpu.VMEM((_N, _COUT), jnp.float32)],
        compiler_params=pltpu.CompilerParams(allow_input_fusion=[True] * 9),
    )(pos4s, pos4s.T, xcol, xcol.reshape(1, _N), p2s, p2s.reshape(1, _N),
      dTs, w2T, bias_row)


@functools.partial(
    pl.kernel,
    out_type=jax.ShapeDtypeStruct((_N, _COUT), jnp.float32),
    mesh=_mesh,
    scratch_types=[
        pltpu.VMEM((_PPT,), jnp.int32),
        pltpu.VMEM((_PPT, _COUT), jnp.float32),
        pltpu.SemaphoreType.DMA,
    ],
    compiler_params=pltpu.CompilerParams(needs_layout_passes=False, use_tc_tiling_on_sc=False),
)
def _gatherback(outs_hbm, rank_hbm, outT_hbm, idx_v, rows_v, sem):
    wid = lax.axis_index("s")
    base = wid * _PPT
    pltpu.sync_copy(rank_hbm.at[pl.ds(base, _PPT)], idx_v)
    pltpu.async_copy(outs_hbm.at[idx_v], rows_v, sem).wait()
    pltpu.sync_copy(rows_v, outT_hbm.at[pl.ds(base, _PPT)])


def kernel(locs, data, density, weight, bias):
    B, N, _ = locs.shape
    pos = locs[0, :, :3]
    x = pos[:, 0]
    zcol = jnp.zeros((N, 1), jnp.float32)
    p2 = jnp.sum(pos * pos, axis=1, keepdims=True)
    pr = jnp.concatenate(
        [pos, zcol, p2, jnp.zeros((N, 11), jnp.float32)], axis=1)
    dT = (data[0] / density.reshape(1, N)).T
    w2T = weight[:, :, 0].T
    bias_row = bias.reshape(1, _COUT)

    prs, dTs, rank = _binperm(x, pr, dT)
    pos4s = prs[:, :4]
    outs = _windowed(pos4s, prs[:, 0:1], prs[:, 4:5], dTs, w2T, bias_row)
    outT = _gatherback(outs, rank)
    return outT.T.reshape(B, _COUT, N)
